# baseline (speedup 1.0000x reference)
import functools

import jax
import jax.numpy as jnp
from jax import lax
from jax.experimental import pallas as pl
from jax.experimental.pallas import tpu as pltpu
from jax.experimental.pallas import tpu_sc as plsc

_BIG = 1e30
_PAD = 1e17
_L = 16
_NSC = 2

_GDN = lax.GatherDimensionNumbers(
    offset_dims=(), collapsed_slice_dims=(0,), start_index_map=(0,))


def _splat(vec, l):
    idx = jnp.full((_L, 1), l, jnp.int32)
    return lax.gather(vec, idx, _GDN, (1,),
                      mode=lax.GatherScatterMode.PROMISE_IN_BOUNDS)


def _perm(vec, idx):
    return lax.gather(vec, idx.reshape(_L, 1), _GDN, (1,),
                      mode=lax.GatherScatterMode.PROMISE_IN_BOUNDS)


def _vmin16(v):
    ln = jnp.arange(_L, dtype=jnp.int32)
    for step in (8, 4, 2, 1):
        v = jnp.minimum(v, _perm(v, ln ^ step))
    return v


def _vmax16(v):
    ln = jnp.arange(_L, dtype=jnp.int32)
    for step in (8, 4, 2, 1):
        v = jnp.maximum(v, _perm(v, ln ^ step))
    return v


def _pair_hausdorff_sq(v1v, v2v, minqv, n1, n2, lanes):
    n1c = (n1 + (_L - 1)) // _L
    n2g = (n2 + (2 * _L - 1)) // (2 * _L)
    bigv = jnp.full((_L,), _BIG, jnp.float32)

    def init_q(qg, c):
        minqv[pl.ds(qg * 2 * _L, _L)] = bigv
        minqv[pl.ds(qg * 2 * _L + _L, _L)] = bigv
        return c

    lax.fori_loop(0, n2g, init_q, 0)

    def p_loop(pc, h1):
        px = v1v[0, pl.ds(pc * _L, _L)]
        py = v1v[1, pl.ds(pc * _L, _L)]
        pz = v1v[2, pl.ds(pc * _L, _L)]

        @plsc.parallel_loop(0, n2g, 1, unroll=2, carry=(bigv,) * _L)
        def q_loop(qg, accs):
            qa = qg * 2 * _L
            qb = qa + _L
            qx0 = v2v[0, pl.ds(qa, _L)]
            qy0 = v2v[1, pl.ds(qa, _L)]
            qz0 = v2v[2, pl.ds(qa, _L)]
            qx1 = v2v[0, pl.ds(qb, _L)]
            qy1 = v2v[1, pl.ds(qb, _L)]
            qz1 = v2v[2, pl.ds(qb, _L)]
            mq0 = minqv[pl.ds(qa, _L)]
            mq1 = minqv[pl.ds(qb, _L)]
            new_accs = []
            for l in range(_L):
                sx = _splat(px, l)
                sy = _splat(py, l)
                sz = _splat(pz, l)
                dx0 = qx0 - sx
                dy0 = qy0 - sy
                dz0 = qz0 - sz
                d20 = dx0 * dx0 + dy0 * dy0 + dz0 * dz0
                dx1 = qx1 - sx
                dy1 = qy1 - sy
                dz1 = qz1 - sz
                d21 = dx1 * dx1 + dy1 * dy1 + dz1 * dz1
                mq0 = jnp.minimum(mq0, d20)
                mq1 = jnp.minimum(mq1, d21)
                new_accs.append(jnp.minimum(accs[l], jnp.minimum(d20, d21)))
            minqv[pl.ds(qa, _L)] = mq0
            minqv[pl.ds(qb, _L)] = mq1
            return tuple(new_accs)

        accs = q_loop
        zerov = jnp.zeros((_L,), jnp.float32)
        for l in range(_L):
            vm = _vmin16(accs[l])
            h1 = jnp.maximum(h1, jnp.where(pc * _L + l < n1, vm, zerov))
        return h1

    h1v = lax.fori_loop(0, n1c, p_loop, jnp.zeros((_L,), jnp.float32))

    def h2_loop(qc, h2v):
        mq = minqv[pl.ds(qc * _L, _L)]
        return jnp.maximum(h2v, jnp.where(lanes + qc * _L < n2, mq, -_BIG))

    n2c = jnp.where(n1 > 0, (n2 + (_L - 1)) // _L, 0)
    h2v = lax.fori_loop(0, n2c, h2_loop, jnp.full((_L,), -_BIG, jnp.float32))
    return jnp.maximum(h1v, jnp.maximum(_vmax16(h2v), 0.0))


def _sc_body(v1_hbm, v2_hbm, prm_hbm, out_hbm, v1v, v2v, prmv, minqv, outv):
    wid = lax.axis_index("c") * 16 + lax.axis_index("s")
    pltpu.sync_copy(prm_hbm.at[wid], prmv)
    lanes = lax.iota(jnp.int32, _L)
    prm = prmv[...]
    i = prm[0].astype(jnp.int32)
    j = prm[1].astype(jnp.int32)
    n1 = prm[2].astype(jnp.int32)
    n2 = prm[3].astype(jnp.int32)
    pltpu.sync_copy(v1_hbm.at[i], v1v)
    pltpu.sync_copy(v2_hbm.at[j], v2v)
    h = _pair_hausdorff_sq(v1v, v2v, minqv, n1, n2, lanes)
    outv[...] = jnp.where(lanes == 0, h, 0.0)
    pltpu.sync_copy(outv, out_hbm.at[wid])


def _tc_body(tcr_ref, sz1_ref, sz2_ref, v1_ref, v2_ref, out_ref):
    i = pl.program_id(0)
    x = v1_ref[0]
    L1 = x.shape[0]
    B2 = out_ref.shape[1]
    L2 = v2_ref.shape[1]
    yf = v2_ref[...].reshape(B2 * L2, 3)
    x2 = jnp.sum(x * x, axis=1, keepdims=True)
    y2 = jnp.sum(yf * yf, axis=1, keepdims=True)
    yy = jnp.concatenate([yf, y2], axis=1)
    xx = jnp.concatenate([-2.0 * x, jnp.ones((L1, 1), jnp.float32)], axis=1)
    g = lax.dot_general(xx, yy, (((1,), (1,)), ((), ())),
                        preferred_element_type=jnp.float32)
    d2 = x2 + g

    n1 = sz1_ref[tcr_ref[i]]
    rmask = lax.broadcasted_iota(jnp.int32, (L1, 1), 0) < n1
    cios = lax.broadcasted_iota(jnp.int32, (1, L2), 1)
    for j in range(B2):
        dj = d2[:, j * L2:(j + 1) * L2]
        n2 = sz2_ref[j]
        minq = jnp.min(dj, axis=1, keepdims=True)
        minp = jnp.min(dj, axis=0, keepdims=True)
        h1 = jnp.max(jnp.where(rmask, minq, -_BIG))
        h2 = jnp.max(jnp.where(cios < n2, minp, -_BIG))
        out_ref[0, j] = jnp.full((8, 128), jnp.maximum(h1, h2), jnp.float32)


def kernel(v1, sz1, v2, sz2):
    B1, L1, _ = v1.shape
    B2, L2, _ = v2.shape
    nw = 32
    ntc = B1 - _NSC
    sz1 = sz1.astype(jnp.int32)
    sz2 = sz2.astype(jnp.int32)
    m1 = jnp.arange(L1)[None, :, None] < sz1[:, None, None]
    m2 = jnp.arange(L2)[None, :, None] < sz2[:, None, None]
    v1p = jnp.where(m1, v1, _PAD)
    v2p = jnp.where(m2, v2, _PAD)
    order = jnp.argsort(sz1)
    scr = order[:_NSC]
    tcr = order[_NSC:]

    v1t = jnp.transpose(v1p, (0, 2, 1))
    v2t = jnp.transpose(v2p, (0, 2, 1))
    t = jnp.arange(nw, dtype=jnp.int32)
    si = jnp.where(t < _NSC * B2, scr[jnp.minimum(t // B2, _NSC - 1)], 0)
    sj = t % B2
    sn1 = jnp.where(t < _NSC * B2, sz1[si], 0)
    cols = jnp.stack([si, sj, sn1, sz2[sj]], axis=1)
    prm = jnp.pad(cols, ((0, 0), (0, _L - 4))).astype(jnp.float32)

    sc = functools.partial(
        pl.kernel,
        mesh=plsc.VectorSubcoreMesh(core_axis_name="c", subcore_axis_name="s"),
        out_type=jax.ShapeDtypeStruct((nw, _L), jnp.float32),
        scratch_types=[
            pltpu.VMEM((3, L1), jnp.float32),
            pltpu.VMEM((3, L2), jnp.float32),
            pltpu.VMEM((_L,), jnp.float32),
            pltpu.VMEM((L2,), jnp.float32),
            pltpu.VMEM((_L,), jnp.float32),
        ],
    )(_sc_body)
    res_sc = sc(v1t, v2t, prm)
    sc_rows = res_sc[:_NSC * B2, 0].reshape(_NSC, B2)

    tc_out = pl.pallas_call(
        _tc_body,
        grid_spec=pltpu.PrefetchScalarGridSpec(
            num_scalar_prefetch=1,
            grid=(ntc,),
            in_specs=[
                pl.BlockSpec(memory_space=pltpu.SMEM),
                pl.BlockSpec(memory_space=pltpu.SMEM),
                pl.BlockSpec((1, L1, 3), lambda i, tcr_ref: (tcr_ref[i], 0, 0)),
                pl.BlockSpec((B2, L2, 3), lambda i, tcr_ref: (0, 0, 0)),
            ],
            out_specs=pl.BlockSpec(
                (1, B2, 8, 128), lambda i, tcr_ref: (i, 0, 0, 0)),
        ),
        out_shape=jax.ShapeDtypeStruct((ntc, B2, 8, 128), jnp.float32),
    )(tcr.astype(jnp.int32), sz1, sz2, v1p, v2p)
    tc_rows = tc_out[:, :, 0, 0]

    out = jnp.zeros((B1, B2), jnp.float32)
    out = out.at[scr].set(sc_rows).at[tcr].set(tc_rows)
    return jnp.sqrt(jnp.maximum(out, 0.0))


# speedup vs baseline: 1.0868x; 1.0868x over previous
import functools

import jax
import jax.numpy as jnp
from jax import lax
from jax.experimental import pallas as pl
from jax.experimental.pallas import tpu as pltpu
from jax.experimental.pallas import tpu_sc as plsc

_BIG = 1e30
_PAD = 1e17
_L = 16
_NSC = 2

_GDN = lax.GatherDimensionNumbers(
    offset_dims=(), collapsed_slice_dims=(0,), start_index_map=(0,))


def _splat(vec, l):
    idx = jnp.full((_L, 1), l, jnp.int32)
    return lax.gather(vec, idx, _GDN, (1,),
                      mode=lax.GatherScatterMode.PROMISE_IN_BOUNDS)


def _perm(vec, idx):
    return lax.gather(vec, idx.reshape(_L, 1), _GDN, (1,),
                      mode=lax.GatherScatterMode.PROMISE_IN_BOUNDS)


def _vmin16(v):
    ln = jnp.arange(_L, dtype=jnp.int32)
    for step in (8, 4, 2, 1):
        v = jnp.minimum(v, _perm(v, ln ^ step))
    return v


def _vmax16(v):
    ln = jnp.arange(_L, dtype=jnp.int32)
    for step in (8, 4, 2, 1):
        v = jnp.maximum(v, _perm(v, ln ^ step))
    return v


def _pair_hausdorff_sq(v1v, v2v, minqv, i, j, n1, n2, lanes):
    n1c = (n1 + (_L - 1)) // _L
    n2g = (n2 + (2 * _L - 1)) // (2 * _L)
    bigv = jnp.full((_L,), _BIG, jnp.float32)

    def init_q(qg, c):
        minqv[pl.ds(qg * 2 * _L, _L)] = bigv
        minqv[pl.ds(qg * 2 * _L + _L, _L)] = bigv
        return c

    lax.fori_loop(0, n2g, init_q, 0)

    def p_loop(pc, h1):
        px = v1v[i, 0, pl.ds(pc * _L, _L)]
        py = v1v[i, 1, pl.ds(pc * _L, _L)]
        pz = v1v[i, 2, pl.ds(pc * _L, _L)]

        @plsc.parallel_loop(0, n2g, 1, unroll=2, carry=(bigv,) * _L)
        def q_loop(qg, accs):
            qa = qg * 2 * _L
            qb = qa + _L
            qx0 = v2v[j, 0, pl.ds(qa, _L)]
            qy0 = v2v[j, 1, pl.ds(qa, _L)]
            qz0 = v2v[j, 2, pl.ds(qa, _L)]
            qx1 = v2v[j, 0, pl.ds(qb, _L)]
            qy1 = v2v[j, 1, pl.ds(qb, _L)]
            qz1 = v2v[j, 2, pl.ds(qb, _L)]
            mq0 = minqv[pl.ds(qa, _L)]
            mq1 = minqv[pl.ds(qb, _L)]
            new_accs = []
            for l in range(_L):
                sx = _splat(px, l)
                sy = _splat(py, l)
                sz = _splat(pz, l)
                dx0 = qx0 - sx
                dy0 = qy0 - sy
                dz0 = qz0 - sz
                d20 = dx0 * dx0 + dy0 * dy0 + dz0 * dz0
                dx1 = qx1 - sx
                dy1 = qy1 - sy
                dz1 = qz1 - sz
                d21 = dx1 * dx1 + dy1 * dy1 + dz1 * dz1
                mq0 = jnp.minimum(mq0, d20)
                mq1 = jnp.minimum(mq1, d21)
                new_accs.append(jnp.minimum(accs[l], jnp.minimum(d20, d21)))
            minqv[pl.ds(qa, _L)] = mq0
            minqv[pl.ds(qb, _L)] = mq1
            return tuple(new_accs)

        accs = q_loop
        zerov = jnp.zeros((_L,), jnp.float32)
        for l in range(_L):
            vm = _vmin16(accs[l])
            h1 = jnp.maximum(h1, jnp.where(pc * _L + l < n1, vm, zerov))
        return h1

    h1v = lax.fori_loop(0, n1c, p_loop, jnp.zeros((_L,), jnp.float32))

    def h2_loop(qc, h2v):
        mq = minqv[pl.ds(qc * _L, _L)]
        return jnp.maximum(h2v, jnp.where(lanes + qc * _L < n2, mq, -_BIG))

    n2c = jnp.where(n1 > 0, (n2 + (_L - 1)) // _L, 0)
    h2v = lax.fori_loop(0, n2c, h2_loop, jnp.full((_L,), -_BIG, jnp.float32))
    return jnp.maximum(h1v, jnp.maximum(_vmax16(h2v), 0.0))


def _sc_body(v1_hbm, v2_hbm, prm_hbm, out_hbm, v1v, v2v, prmv, minqv, outv):
    wid = lax.axis_index("c") * 16 + lax.axis_index("s")
    pltpu.sync_copy(v1_hbm, v1v)
    pltpu.sync_copy(v2_hbm, v2v)
    pltpu.sync_copy(prm_hbm.at[wid], prmv)
    lanes = lax.iota(jnp.int32, _L)
    prm = prmv[...]
    i = prm[0].astype(jnp.int32)
    j = prm[1].astype(jnp.int32)
    n1 = prm[2].astype(jnp.int32)
    n2 = prm[3].astype(jnp.int32)
    h = _pair_hausdorff_sq(v1v, v2v, minqv, i, j, n1, n2, lanes)
    outv[...] = jnp.where(lanes == 0, h, 0.0)
    pltpu.sync_copy(outv, out_hbm.at[wid])


def _tc_body(sz1_ref, sz2_ref, v1_ref, v2_ref, out_ref):
    i = pl.program_id(0)
    x = v1_ref[0]
    yf = v2_ref[...]
    L1 = x.shape[0]
    B2 = out_ref.shape[1]
    L2 = yf.shape[0] // B2
    x2 = jnp.sum(x * x, axis=1, keepdims=True)
    y2 = jnp.sum(yf * yf, axis=1, keepdims=True)
    yy = jnp.concatenate([yf, y2], axis=1)
    xx = jnp.concatenate([-2.0 * x, jnp.ones((L1, 1), jnp.float32)], axis=1)
    g = lax.dot_general(xx, yy, (((1,), (1,)), ((), ())),
                        preferred_element_type=jnp.float32)
    d2 = x2 + g

    n1 = sz1_ref[i]
    rmask = lax.broadcasted_iota(jnp.int32, (L1, 1), 0) < n1
    cios = lax.broadcasted_iota(jnp.int32, (1, L2), 1)
    for j in range(B2):
        dj = d2[:, j * L2:(j + 1) * L2]
        n2 = sz2_ref[j]
        minq = jnp.min(dj, axis=1, keepdims=True)
        minp = jnp.min(dj, axis=0, keepdims=True)
        h1 = jnp.max(jnp.where(rmask, minq, -_BIG))
        h2 = jnp.max(jnp.where(cios < n2, minp, -_BIG))
        out_ref[0, j] = jnp.full((8, 128), jnp.maximum(h1, h2), jnp.float32)


def kernel(v1, sz1, v2, sz2):
    B1, L1, _ = v1.shape
    B2, L2, _ = v2.shape
    nw = 32
    ntc = B1 - _NSC
    sz1 = sz1.astype(jnp.int32)
    sz2 = sz2.astype(jnp.int32)
    m1 = jnp.arange(L1)[None, :, None] < sz1[:, None, None]
    m2 = jnp.arange(L2)[None, :, None] < sz2[:, None, None]
    v1p = jnp.where(m1, v1, _PAD)
    v2p = jnp.where(m2, v2, _PAD)
    order = jnp.argsort(sz1)
    scr = order[:_NSC]
    tcr = order[_NSC:]

    v1t = jnp.transpose(v1p, (0, 2, 1))
    v2t = jnp.transpose(v2p, (0, 2, 1))
    t = jnp.arange(nw, dtype=jnp.int32)
    si = jnp.where(t < _NSC * B2, scr[jnp.minimum(t // B2, _NSC - 1)], 0)
    sj = t % B2
    sn1 = jnp.where(t < _NSC * B2, sz1[si], 0)
    cols = jnp.stack([si, sj, sn1, sz2[sj]], axis=1)
    prm = jnp.pad(cols, ((0, 0), (0, _L - 4))).astype(jnp.float32)

    sc = functools.partial(
        pl.kernel,
        mesh=plsc.VectorSubcoreMesh(core_axis_name="c", subcore_axis_name="s"),
        out_type=jax.ShapeDtypeStruct((nw, _L), jnp.float32),
        scratch_types=[
            pltpu.VMEM((B1, 3, L1), jnp.float32),
            pltpu.VMEM((B2, 3, L2), jnp.float32),
            pltpu.VMEM((_L,), jnp.float32),
            pltpu.VMEM((L2,), jnp.float32),
            pltpu.VMEM((_L,), jnp.float32),
        ],
    )(_sc_body)
    res_sc = sc(v1t, v2t, prm)
    sc_rows = res_sc[:_NSC * B2, 0].reshape(_NSC, B2)

    v1g = v1p[tcr]
    tc_out = pl.pallas_call(
        _tc_body,
        grid=(ntc,),
        in_specs=[
            pl.BlockSpec(memory_space=pltpu.SMEM),
            pl.BlockSpec(memory_space=pltpu.SMEM),
            pl.BlockSpec((1, L1, 3), lambda i: (i, 0, 0)),
            pl.BlockSpec((B2 * L2, 3), lambda i: (0, 0)),
        ],
        out_specs=pl.BlockSpec((1, B2, 8, 128), lambda i: (i, 0, 0, 0)),
        out_shape=jax.ShapeDtypeStruct((ntc, B2, 8, 128), jnp.float32),
    )(sz1[tcr], sz2, v1g, v2p.reshape(B2 * L2, 3))
    tc_rows = tc_out[:, :, 0, 0]

    out = jnp.zeros((B1, B2), jnp.float32)
    out = out.at[scr].set(sc_rows).at[tcr].set(tc_rows)
    return jnp.sqrt(jnp.maximum(out, 0.0))
